# baseline (speedup 1.0000x reference)
import jax
import jax.numpy as jnp
from jax.experimental import pallas as pl

_C_BLK = 8


def _broadcast_body(in_ref, out_ref):
    blk = in_ref[...]
    for b in range(out_ref.shape[0]):
        out_ref[b] = blk


def kernel(input, famp):
    B = input.shape[0]
    _, C, H, W = famp.shape
    src = famp.reshape(C, H, W)
    out = pl.pallas_call(
        _broadcast_body,
        grid=(C // _C_BLK,),
        in_specs=[pl.BlockSpec((_C_BLK, H, W), lambda i: (i, 0, 0))],
        out_specs=pl.BlockSpec((B, _C_BLK, H, W), lambda i: (0, i, 0, 0)),
        out_shape=jax.ShapeDtypeStruct((B, C, H, W), famp.dtype),
    )(src)
    return out
